# baseline (speedup 1.0000x reference)
import jax
import jax.numpy as jnp
from jax.experimental import pallas as pl
from jax.experimental.pallas import tpu as pltpu

_B, _R, _C = 32, 4096, 1024
_NROWS = _B * _R
_BLOCK_ROWS = 2048
_BLOCK = _BLOCK_ROWS * _C

_KS0 = 0
_KS1 = 42
_KS2 = 0x1BD11BDA ^ _KS0 ^ _KS1
_ROT0 = (13, 15, 26, 6)
_ROT1 = (17, 29, 16, 24)


def _rotl(x, r):
    return (x << jnp.uint32(r)) | (x >> jnp.uint32(32 - r))


def _threefry_bits(x1):
    ks0 = jnp.uint32(_KS0)
    ks1 = jnp.uint32(_KS1)
    ks2 = jnp.uint32(_KS2)
    x0 = ks0

    def rounds(x0, x1, rots):
        for r in rots:
            x0 = x0 + x1
            x1 = _rotl(x1, r)
            x1 = x0 ^ x1
        return x0, x1

    x0, x1 = rounds(x0, x1, _ROT0)
    x0 = x0 + ks1
    x1 = x1 + (ks2 + jnp.uint32(1))
    x0, x1 = rounds(x0, x1, _ROT1)
    x0 = x0 + ks2
    x1 = x1 + (ks0 + jnp.uint32(2))
    x0, x1 = rounds(x0, x1, _ROT0)
    x0 = x0 + ks0
    x1 = x1 + (ks1 + jnp.uint32(3))
    x0, x1 = rounds(x0, x1, _ROT1)
    x0 = x0 + ks1
    x1 = x1 + (ks2 + jnp.uint32(4))
    x0, x1 = rounds(x0, x1, _ROT0)
    x0 = x0 + ks2
    x1 = x1 + (ks0 + jnp.uint32(5))
    return x0 ^ x1


def _body(lin_ref, x_ref, o_ref):
    g = pl.program_id(0)
    off = g.astype(jnp.uint32) * jnp.uint32(_BLOCK)
    x1 = lin_ref[...] + off
    bits = _threefry_bits(x1)
    u = pltpu.bitcast((bits >> jnp.uint32(9)) | jnp.uint32(0x3F800000),
                      jnp.float32) - jnp.float32(1.0)
    o_ref[...] = x_ref[...] + u


def kernel(x):
    x2 = x.reshape(_NROWS, _C)
    lin = (jnp.arange(_BLOCK, dtype=jnp.uint32) + jnp.uint32(_KS1)).reshape(
        _BLOCK_ROWS, _C)
    out = pl.pallas_call(
        _body,
        grid=(_NROWS // _BLOCK_ROWS,),
        in_specs=[
            pl.BlockSpec((_BLOCK_ROWS, _C), lambda g: (0, 0)),
            pl.BlockSpec((_BLOCK_ROWS, _C), lambda g: (g, 0)),
        ],
        out_specs=pl.BlockSpec((_BLOCK_ROWS, _C), lambda g: (g, 0)),
        out_shape=jax.ShapeDtypeStruct((_NROWS, _C), jnp.float32),
        compiler_params=pltpu.CompilerParams(
            dimension_semantics=("arbitrary",),
            vmem_limit_bytes=50 * 1024 * 1024),
    )(lin, x2)
    return out.reshape(_B, _R, _C)


# speedup vs baseline: 1.0475x; 1.0475x over previous
import jax
import jax.numpy as jnp
from jax.experimental import pallas as pl
from jax.experimental.pallas import tpu as pltpu

_B, _R, _C = 32, 4096, 1024
_NROWS = _B * _R
_BLOCK_ROWS = 1024
_BLOCK = _BLOCK_ROWS * _C
_TAIL = _BLOCK_ROWS - 8

_KS0 = 0
_KS1 = 42
_KS2 = 0x1BD11BDA ^ _KS0 ^ _KS1
_ROT1 = (17, 29, 16, 24)


def _rotl(x, r):
    return (x << jnp.uint32(r)) | (x >> jnp.uint32(32 - r))


def _finish_from_r2(x0, x1):
    ks0 = jnp.uint32(_KS0)
    ks1 = jnp.uint32(_KS1)
    ks2 = jnp.uint32(_KS2)

    def rounds(x0, x1, rots):
        for r in rots:
            x0 = x0 + x1
            x1 = _rotl(x1, r)
            x1 = x0 ^ x1
        return x0, x1

    x0, x1 = rounds(x0, x1, (26, 6))
    x0 = x0 + ks1
    x1 = x1 + (ks2 + jnp.uint32(1))
    x0, x1 = rounds(x0, x1, _ROT1)
    x0 = x0 + ks2
    x1 = x1 + (ks0 + jnp.uint32(2))
    x0, x1 = rounds(x0, x1, (13, 15, 26, 6))
    x0 = x0 + ks0
    x1 = x1 + (ks1 + jnp.uint32(3))
    x0, x1 = rounds(x0, x1, _ROT1)
    x0 = x0 + ks1
    x1 = x1 + (ks2 + jnp.uint32(4))
    x0, x1 = rounds(x0, x1, (13, 15, 26, 6))
    x0 = x0 + ks2
    x1 = x1 + (ks0 + jnp.uint32(5))
    return x0 ^ x1


def _unit(bits):
    return pltpu.bitcast((bits >> jnp.uint32(9)) | jnp.uint32(0x3F800000),
                         jnp.float32) - jnp.float32(1.0)


def _body(x_ref, o_ref, lin2_ref, p1_ref, p2_ref):
    g = pl.program_id(0)

    @pl.when(g == 0)
    def _():
        ri = jax.lax.broadcasted_iota(jnp.uint32, (_BLOCK_ROWS, _C), 0)
        ci = jax.lax.broadcasted_iota(jnp.uint32, (_BLOCK_ROWS, _C), 1)
        lin2 = (ri << jnp.uint32(10)) + ci + jnp.uint32(_KS1)
        p1 = _rotl(lin2, 13) ^ lin2
        lin2_ref[...] = lin2
        p1_ref[...] = p1
        p2_ref[...] = _rotl(p1, 15)

    off = g.astype(jnp.uint32) * jnp.uint32(_BLOCK)
    q1 = _rotl(off, 13) ^ off
    q2 = _rotl(q1, 15)

    lin2 = lin2_ref[...]
    x1_0 = lin2 + off
    x1_1 = p1_ref[...] ^ q1
    x0_2 = x1_0 + x1_1
    x1_2 = (p2_ref[...] ^ x0_2) ^ q2
    o_ref[...] = x_ref[...] + _unit(_finish_from_r2(x0_2, x1_2))

    lt = lin2_ref[_TAIL:, :]
    t1_0 = lt + off
    t1_1 = _rotl(t1_0, 13) ^ t1_0
    t0_2 = t1_0 + t1_1
    t1_2 = _rotl(t1_1, 15) ^ t0_2
    o_ref[_TAIL:, :] = x_ref[_TAIL:, :] + _unit(_finish_from_r2(t0_2, t1_2))


def kernel(x):
    x2 = x.reshape(_NROWS, _C)
    out = pl.pallas_call(
        _body,
        grid=(_NROWS // _BLOCK_ROWS,),
        in_specs=[pl.BlockSpec((_BLOCK_ROWS, _C), lambda g: (g, 0))],
        out_specs=pl.BlockSpec((_BLOCK_ROWS, _C), lambda g: (g, 0)),
        out_shape=jax.ShapeDtypeStruct((_NROWS, _C), jnp.float32),
        scratch_shapes=[
            pltpu.VMEM((_BLOCK_ROWS, _C), jnp.uint32),
            pltpu.VMEM((_BLOCK_ROWS, _C), jnp.uint32),
            pltpu.VMEM((_BLOCK_ROWS, _C), jnp.uint32),
        ],
        compiler_params=pltpu.CompilerParams(
            dimension_semantics=("arbitrary",),
            vmem_limit_bytes=50 * 1024 * 1024),
    )(x2)
    return out.reshape(_B, _R, _C)


# speedup vs baseline: 1.0482x; 1.0007x over previous
import jax
import jax.numpy as jnp
from jax.experimental import pallas as pl
from jax.experimental.pallas import tpu as pltpu

_B, _R, _C = 32, 4096, 1024
_NROWS = _B * _R
_BLOCK_ROWS = 1024
_BLOCK = _BLOCK_ROWS * _C
_TAIL = _BLOCK_ROWS - 8
_TAILC = _C - 128

_KS0 = 0
_KS1 = 42
_KS2 = 0x1BD11BDA ^ _KS0 ^ _KS1
_ROT1 = (17, 29, 16, 24)


def _rotl(x, r):
    return (x << jnp.uint32(r)) | (x >> jnp.uint32(32 - r))


def _finish_from_r2(x0, x1):
    ks0 = jnp.uint32(_KS0)
    ks1 = jnp.uint32(_KS1)
    ks2 = jnp.uint32(_KS2)

    def rounds(x0, x1, rots):
        for r in rots:
            x0 = x0 + x1
            x1 = _rotl(x1, r)
            x1 = x0 ^ x1
        return x0, x1

    x0, x1 = rounds(x0, x1, (26, 6))
    x0 = x0 + ks1
    x1 = x1 + (ks2 + jnp.uint32(1))
    x0, x1 = rounds(x0, x1, _ROT1)
    x0 = x0 + ks2
    x1 = x1 + (ks0 + jnp.uint32(2))
    x0, x1 = rounds(x0, x1, (13, 15, 26, 6))
    x0 = x0 + ks0
    x1 = x1 + (ks1 + jnp.uint32(3))
    x0, x1 = rounds(x0, x1, _ROT1)
    x0 = x0 + ks1
    x1 = x1 + (ks2 + jnp.uint32(4))
    x0, x1 = rounds(x0, x1, (13, 15, 26, 6))
    x0 = x0 + ks2
    x1 = x1 + (ks0 + jnp.uint32(5))
    return x0 ^ x1


def _unit(bits):
    return pltpu.bitcast((bits >> jnp.uint32(9)) | jnp.uint32(0x3F800000),
                         jnp.float32) - jnp.float32(1.0)


def _body(x_ref, o_ref, lin2_ref, p1_ref, p2_ref):
    g = pl.program_id(0)

    @pl.when(g == 0)
    def _():
        ri = jax.lax.broadcasted_iota(jnp.uint32, (_BLOCK_ROWS, _C), 0)
        ci = jax.lax.broadcasted_iota(jnp.uint32, (_BLOCK_ROWS, _C), 1)
        lin2 = (ri << jnp.uint32(10)) + ci + jnp.uint32(_KS1)
        p1 = _rotl(lin2, 13) ^ lin2
        lin2_ref[...] = lin2
        p1_ref[...] = p1
        p2_ref[...] = _rotl(p1, 15)

    off = g.astype(jnp.uint32) * jnp.uint32(_BLOCK)
    q1 = _rotl(off, 13) ^ off
    q2 = _rotl(q1, 15)

    lin2 = lin2_ref[...]
    x1_0 = lin2 + off
    x1_1 = p1_ref[...] ^ q1
    x0_2 = x1_0 + x1_1
    x1_2 = (p2_ref[...] ^ x0_2) ^ q2
    o_ref[...] = x_ref[...] + _unit(_finish_from_r2(x0_2, x1_2))

    lt = lin2_ref[_TAIL:, _TAILC:]
    t1_0 = lt + off
    t1_1 = _rotl(t1_0, 13) ^ t1_0
    t0_2 = t1_0 + t1_1
    t1_2 = _rotl(t1_1, 15) ^ t0_2
    o_ref[_TAIL:, _TAILC:] = (
        x_ref[_TAIL:, _TAILC:] + _unit(_finish_from_r2(t0_2, t1_2)))


def kernel(x):
    x2 = x.reshape(_NROWS, _C)
    out = pl.pallas_call(
        _body,
        grid=(_NROWS // _BLOCK_ROWS,),
        in_specs=[pl.BlockSpec((_BLOCK_ROWS, _C), lambda g: (g, 0))],
        out_specs=pl.BlockSpec((_BLOCK_ROWS, _C), lambda g: (g, 0)),
        out_shape=jax.ShapeDtypeStruct((_NROWS, _C), jnp.float32),
        scratch_shapes=[
            pltpu.VMEM((_BLOCK_ROWS, _C), jnp.uint32),
            pltpu.VMEM((_BLOCK_ROWS, _C), jnp.uint32),
            pltpu.VMEM((_BLOCK_ROWS, _C), jnp.uint32),
        ],
        compiler_params=pltpu.CompilerParams(
            dimension_semantics=("arbitrary",),
            vmem_limit_bytes=50 * 1024 * 1024),
    )(x2)
    return out.reshape(_B, _R, _C)
